# baseline (speedup 1.0000x reference)
import jax
import jax.numpy as jnp
from jax import lax
from jax.experimental import pallas as pl
from jax.experimental.pallas import tpu as pltpu
from jax.experimental.pallas import tpu_sc as plsc

BATCH = 16384
NROWS = 1000000
D0, D1, D2 = 16, 32, 64

_NC = 2
_NS = 16
_NW = _NC * _NS
_BPW = BATCH // _NW
_CHUNK = 128
_NCHUNK = _BPW // _CHUNK

_TABLES = (
    (D0, 3),
    (D1, 2),
    (D2, 1),
)


def _sc_body(idx_hbm, w0, w1, w2, o0, o1, o2,
             idx_v, g0, g1, g2, f0, f1, f2, gbuf, obuf,
             sa, sb, soa, sob):
    wid = lax.axis_index("s") * _NC + lax.axis_index("c")
    base = wid * _BPW

    pltpu.sync_copy(idx_hbm.at[pl.ds(base, _BPW)], idx_v)

    for gref, offr, (d, sh) in ((g0, f0, _TABLES[0]), (g1, f1, _TABLES[1]),
                                (g2, f2, _TABLES[2])):
        mask = (1 << sh) - 1
        dlog = d.bit_length() - 1
        for s in range(_BPW // 16):
            v = idx_v[pl.ds(s * 16, 16)]
            dst = (s // 8, pl.ds((s % 8) * 16, 16))
            gref[dst[0], dst[1]] = lax.shift_right_logical(v, sh)
            offr[dst[0], dst[1]] = lax.shift_left(v & mask, dlog)

    work = []
    for (w, gref, offr, o, (d, _)) in (
            (w0, g0, f0, o0, _TABLES[0]),
            (w1, g1, f1, o1, _TABLES[1]),
            (w2, g2, f2, o2, _TABLES[2])):
        for j in range(_NCHUNK):
            work.append((w, gref, offr, o, d, j))

    gsems = (sa, sb)
    osems = (soa, sob)
    iota = lax.iota(jnp.int32, 16)

    def fire(item, slot):
        w, gref, _, _, _, j = item
        return pltpu.async_copy(w.at[gref.at[j]], gbuf.at[slot], gsems[slot])

    def extract(item, slot):
        w, gref, offr, o, d, j = item
        nsl = d // 16

        def body(k, carry):
            kk = jnp.full((16,), k, jnp.int32)
            off = plsc.load_gather(offr.at[j], [kk])
            col = off + iota
            for c in range(nsl):
                v = plsc.load_gather(gbuf.at[slot], [kk, col + c * 16])
                obuf[slot, pl.ds(k * d + c * 16, 16)] = jnp.maximum(v, 0.0)
            return carry

        lax.fori_loop(0, _CHUNK, body, 0)
        return pltpu.async_copy(
            obuf.at[slot, pl.ds(0, _CHUNK * d)],
            o.at[pl.ds((base + j * _CHUNK) * d, _CHUNK * d)], osems[slot])

    copies = [fire(work[0], 0), fire(work[1], 1)]
    outs = [None, None]
    for n, item in enumerate(work):
        slot = n % 2
        copies[n].wait()
        if outs[slot] is not None:
            outs[slot].wait()
        outs[slot] = extract(item, slot)
        if n + 2 < len(work):
            copies.append(fire(work[n + 2], slot))
    outs[0].wait()
    outs[1].wait()


_gather_relu = pl.kernel(
    _sc_body,
    out_type=(
        jax.ShapeDtypeStruct((BATCH * D0,), jnp.float32),
        jax.ShapeDtypeStruct((BATCH * D1,), jnp.float32),
        jax.ShapeDtypeStruct((BATCH * D2,), jnp.float32),
    ),
    mesh=plsc.VectorSubcoreMesh(core_axis_name="c", subcore_axis_name="s"),
    compiler_params=pltpu.CompilerParams(
        use_tc_tiling_on_sc=True, needs_layout_passes=False),
    scratch_types=[
        pltpu.VMEM((_BPW,), jnp.int32),
        pltpu.VMEM((_NCHUNK, _CHUNK), jnp.int32),
        pltpu.VMEM((_NCHUNK, _CHUNK), jnp.int32),
        pltpu.VMEM((_NCHUNK, _CHUNK), jnp.int32),
        pltpu.VMEM((_NCHUNK, _CHUNK), jnp.int32),
        pltpu.VMEM((_NCHUNK, _CHUNK), jnp.int32),
        pltpu.VMEM((_NCHUNK, _CHUNK), jnp.int32),
        pltpu.VMEM((2, _CHUNK, 128), jnp.float32),
        pltpu.VMEM((2, _CHUNK * D2), jnp.float32),
        pltpu.SemaphoreType.DMA,
        pltpu.SemaphoreType.DMA,
        pltpu.SemaphoreType.DMA,
        pltpu.SemaphoreType.DMA,
    ],
)


def kernel(indices, W0, W1, W2):
    idx = indices.astype(jnp.int32)
    p0 = W0.reshape(NROWS * D0 // 128, 128)
    p1 = W1.reshape(NROWS * D1 // 128, 128)
    p2 = W2.reshape(NROWS * D2 // 128, 128)
    f0, f1, f2 = _gather_relu(idx, p0, p1, p2)
    return (f0.reshape(BATCH, D0), f1.reshape(BATCH, D1),
            f2.reshape(BATCH, D2))
